# Initial kernel scaffold; baseline (speedup 1.0000x reference)
import functools

import jax
import jax.numpy as jnp
from jax import lax
from jax.experimental import pallas as pl
from jax.experimental.pallas import tpu as pltpu
from jax.experimental.pallas import tpu_sc as plsc

C_IN = 8
H = 16
W = 16
KH = 3
KW = 3
OC = 16
K = 2
HO = H - KH + 1
WO = W - KW + 1
S = HO * WO
N_RF = C_IN * KH * KW
B = 32

NC = 2
NS = 16
NW = NC * NS
S_PER_W = (S + NW - 1) // NW
S_PAD = S_PER_W * NW

XLEN = B * C_IN * H * W
CI_W = S_PER_W * N_RF * K * OC
WT_W = S_PER_W * N_RF * 4 * OC
OUT_W = S_PER_W * B * OC

_mesh = plsc.VectorSubcoreMesh(core_axis_name="c", subcore_axis_name="s")


@functools.partial(
    pl.kernel,
    mesh=_mesh,
    out_type=jax.ShapeDtypeStruct((S_PAD * B * OC,), jnp.float32),
    scratch_types=[
        pltpu.VMEM((XLEN,), jnp.float32),
        pltpu.VMEM((CI_W,), jnp.int32),
        pltpu.VMEM((WT_W,), jnp.float32),
        pltpu.VMEM((OUT_W,), jnp.float32),
    ],
)
def _lutconv_sc(x_hbm, ci_hbm, wt_hbm, out_hbm, x_v, ci_v, wt_v, o_v):
    wid = lax.axis_index("s") * NC + lax.axis_index("c")
    pltpu.sync_copy(x_hbm, x_v)
    pltpu.sync_copy(ci_hbm.at[pl.ds(wid * CI_W, CI_W)], ci_v)
    pltpu.sync_copy(wt_hbm.at[pl.ds(wid * WT_W, WT_W)], wt_v)

    zero = jnp.zeros((OC,), jnp.float32)

    for si in range(S_PER_W):
        ci_base0 = si * N_RF * K * OC
        wt_base0 = si * N_RF * 4 * OC

        def rf_body(rf, carry, ci_base0=ci_base0, wt_base0=wt_base0):
            accs = carry
            cib = ci_base0 + rf * (K * OC)
            wtb = wt_base0 + rf * (4 * OC)
            ci0 = ci_v[pl.ds(cib, OC)]
            ci1 = ci_v[pl.ds(cib + OC, OC)]
            w0 = wt_v[pl.ds(wtb, OC)]
            w1 = wt_v[pl.ds(wtb + OC, OC)]
            w2 = wt_v[pl.ds(wtb + 2 * OC, OC)]
            w3 = wt_v[pl.ds(wtb + 3 * OC, OC)]
            bb = w2 - w0
            cc = w1 - w0
            aa = (w3 + w0) - (w1 + w2)
            out = [None] * (B + 1)
            out[B] = accs[B] + w0
            for b in range(B):
                off = b * (C_IN * H * W)
                p0 = plsc.load_gather(x_v, [ci0 + off])
                p1 = plsc.load_gather(x_v, [ci1 + off])
                out[b] = accs[b] + (p0 * bb + (p1 * cc + (p0 * p1) * aa))
            return tuple(out)

        init = tuple([zero] * (B + 1))
        accs = lax.fori_loop(0, N_RF, rf_body, init)
        sw0 = accs[B]
        for b in range(B):
            o_v[pl.ds((si * B + b) * OC, OC)] = accs[b] + sw0

    pltpu.sync_copy(o_v, out_hbm.at[pl.ds(wid * OUT_W, OUT_W)])


def kernel(x, input_mask, lut_weights):
    xf = x.reshape(-1)
    flat = (input_mask[:, 0] * (H * W) + input_mask[:, 1] * W
            + input_mask[:, 2]).astype(jnp.int32)
    cidx = flat.reshape(OC, S, N_RF, K).transpose(1, 2, 3, 0)
    wt = lut_weights.reshape(OC, S, N_RF, 4).transpose(1, 2, 3, 0)
    pad = S_PAD - S
    cidx = jnp.concatenate(
        [cidx, jnp.zeros((pad,) + cidx.shape[1:], cidx.dtype)], axis=0)
    wt = jnp.concatenate(
        [wt, jnp.zeros((pad,) + wt.shape[1:], wt.dtype)], axis=0)
    out = _lutconv_sc(xf, cidx.reshape(-1), wt.reshape(-1))
    out = out.reshape(S_PAD, B, OC)[:S]
    return out.transpose(1, 2, 0).reshape(B, OC, HO, WO)


# speedup vs baseline: 11.6046x; 11.6046x over previous
import functools

import jax
import jax.numpy as jnp
from jax import lax
from jax.experimental import pallas as pl
from jax.experimental.pallas import tpu as pltpu
from jax.experimental.pallas import tpu_sc as plsc

C_IN = 8
H = 16
W = 16
KH = 3
KW = 3
OC = 16
K = 2
HO = H - KH + 1
WO = W - KW + 1
S = HO * WO
N_RF = C_IN * KH * KW
B = 32

NC = 2
NS = 16
NW = NC * NS
S_PER_W = (S + NW - 1) // NW
S_PAD = S_PER_W * NW

XLEN = B * C_IN * H * W
CI_W = S_PER_W * N_RF * K * OC
WT_W = S_PER_W * N_RF * 4 * OC
OUT_W = S_PER_W * B * OC

_mesh = plsc.VectorSubcoreMesh(core_axis_name="c", subcore_axis_name="s")


@functools.partial(
    pl.kernel,
    mesh=_mesh,
    compiler_params=pltpu.CompilerParams(needs_layout_passes=False),
    out_type=jax.ShapeDtypeStruct((S_PAD * B * OC,), jnp.float32),
    scratch_types=[
        pltpu.VMEM((XLEN,), jnp.float32),
        pltpu.VMEM((CI_W,), jnp.int32),
        pltpu.VMEM((WT_W,), jnp.float32),
        pltpu.VMEM((OUT_W,), jnp.float32),
    ],
)
def _lutconv_sc(x_hbm, ci_hbm, wt_hbm, out_hbm, x_v, ci_v, wt_v, o_v):
    wid = lax.axis_index("s") * NC + lax.axis_index("c")
    pltpu.sync_copy(x_hbm, x_v)
    pltpu.sync_copy(ci_hbm.at[pl.ds(wid * CI_W, CI_W)], ci_v)
    pltpu.sync_copy(wt_hbm.at[pl.ds(wid * WT_W, WT_W)], wt_v)

    zero = jnp.zeros((OC,), jnp.float32)

    for si in range(S_PER_W):
        ci_base0 = si * N_RF * K * OC
        wt_base0 = si * N_RF * 4 * OC
        o_base0 = si * B * OC

        for b in range(B):
            o_v[pl.ds(o_base0 + b * OC, OC)] = zero

        def rf_body(rf, sw0, ci_base0=ci_base0, wt_base0=wt_base0,
                    o_base0=o_base0):
            cib = ci_base0 + rf * (K * OC)
            wtb = wt_base0 + rf * (4 * OC)
            ci0 = ci_v[pl.ds(cib, OC)]
            ci1 = ci_v[pl.ds(cib + OC, OC)]
            w0 = wt_v[pl.ds(wtb, OC)]
            w1 = wt_v[pl.ds(wtb + OC, OC)]
            w2 = wt_v[pl.ds(wtb + 2 * OC, OC)]
            w3 = wt_v[pl.ds(wtb + 3 * OC, OC)]
            bb = w2 - w0
            cc = w1 - w0
            aa = (w3 + w0) - (w1 + w2)
            for b in range(B):
                off = b * (C_IN * H * W)
                p0 = plsc.load_gather(x_v, [ci0 + off])
                p1 = plsc.load_gather(x_v, [ci1 + off])
                v = p0 * bb + (p1 * cc + (p0 * p1) * aa)
                plsc.addupdate(o_v.at[pl.ds(o_base0 + b * OC, OC)], v)
            return sw0 + w0

        sw0 = lax.fori_loop(0, N_RF, rf_body, zero)
        for b in range(B):
            o_v[pl.ds(o_base0 + b * OC, OC)] = (
                o_v[pl.ds(o_base0 + b * OC, OC)] + sw0)

    pltpu.sync_copy(o_v, out_hbm.at[pl.ds(wid * OUT_W, OUT_W)])


def kernel(x, input_mask, lut_weights):
    xf = x.reshape(-1)
    flat = (input_mask[:, 0] * (H * W) + input_mask[:, 1] * W
            + input_mask[:, 2]).astype(jnp.int32)
    cidx = flat.reshape(OC, S, N_RF, K).transpose(1, 2, 3, 0)
    wt = lut_weights.reshape(OC, S, N_RF, 4).transpose(1, 2, 3, 0)
    pad = S_PAD - S
    cidx = jnp.concatenate(
        [cidx, jnp.zeros((pad,) + cidx.shape[1:], cidx.dtype)], axis=0)
    wt = jnp.concatenate(
        [wt, jnp.zeros((pad,) + wt.shape[1:], wt.dtype)], axis=0)
    out = _lutconv_sc(xf, cidx.reshape(-1), wt.reshape(-1))
    out = out.reshape(S_PAD, B, OC)[:S]
    return out.transpose(1, 2, 0).reshape(B, OC, HO, WO)


# speedup vs baseline: 16.6250x; 1.4326x over previous
import functools

import jax
import jax.numpy as jnp
from jax import lax
from jax.experimental import pallas as pl
from jax.experimental.pallas import tpu as pltpu
from jax.experimental.pallas import tpu_sc as plsc

C_IN = 8
H = 16
W = 16
KH = 3
KW = 3
OC = 16
K = 2
HO = H - KH + 1
WO = W - KW + 1
S = HO * WO
N_RF = C_IN * KH * KW
B = 32

NC = 2
NS = 16
NW = 28
S_PER_W = S // NW

XLEN = B * C_IN * H * W
CI_OC = S_PER_W * N_RF * K
WT_OC = S_PER_W * N_RF * 4
OUT_W = S_PER_W * B * OC

_mesh = plsc.VectorSubcoreMesh(core_axis_name="c", subcore_axis_name="s")


@functools.partial(
    pl.kernel,
    mesh=_mesh,
    compiler_params=pltpu.CompilerParams(needs_layout_passes=False),
    out_type=jax.ShapeDtypeStruct((S * B * OC,), jnp.float32),
    scratch_types=[
        pltpu.VMEM((XLEN,), jnp.float32),
        pltpu.VMEM((OC * CI_OC,), jnp.int32),
        pltpu.VMEM((OC * WT_OC,), jnp.float32),
        pltpu.VMEM((OUT_W,), jnp.float32),
    ],
)
def _lutconv_sc(x_hbm, ci_hbm, wt_hbm, out_hbm, x_v, ci_v, wt_v, o_v):
    wid = lax.axis_index("s") * NC + lax.axis_index("c")

    @pl.when(wid < NW)
    def _body():
        pltpu.sync_copy(x_hbm, x_v)
        for oc in range(OC):
            pltpu.sync_copy(
                ci_hbm.at[pl.ds(oc * (S * N_RF * K) + wid * CI_OC, CI_OC)],
                ci_v.at[pl.ds(oc * CI_OC, CI_OC)])
            pltpu.sync_copy(
                wt_hbm.at[pl.ds(oc * (S * N_RF * 4) + wid * WT_OC, WT_OC)],
                wt_v.at[pl.ds(oc * WT_OC, WT_OC)])

        zero = jnp.zeros((OC,), jnp.float32)
        ioc = lax.iota(jnp.int32, OC)
        ioc_ci = ioc * CI_OC
        ioc_wt = ioc * WT_OC

        for si in range(S_PER_W):
            o_base0 = si * B * OC

            for b in range(B):
                o_v[pl.ds(o_base0 + b * OC, OC)] = zero

            def rf_body(rf, sw0, si=si, o_base0=o_base0):
                cib = ioc_ci + (si * (N_RF * K) + rf * K)
                wtb = ioc_wt + (si * (N_RF * 4) + rf * 4)
                ci0 = plsc.load_gather(ci_v, [cib])
                ci1 = plsc.load_gather(ci_v, [cib + 1])
                w0 = plsc.load_gather(wt_v, [wtb])
                w1 = plsc.load_gather(wt_v, [wtb + 1])
                w2 = plsc.load_gather(wt_v, [wtb + 2])
                w3 = plsc.load_gather(wt_v, [wtb + 3])
                bb = w2 - w0
                cc = w1 - w0
                aa = (w3 + w0) - (w1 + w2)
                for b in range(B):
                    off = b * (C_IN * H * W)
                    p0 = plsc.load_gather(x_v, [ci0 + off])
                    p1 = plsc.load_gather(x_v, [ci1 + off])
                    v = p0 * bb + (p1 * cc + (p0 * p1) * aa)
                    plsc.addupdate(o_v.at[pl.ds(o_base0 + b * OC, OC)], v)
                return sw0 + w0

            sw0 = lax.fori_loop(0, N_RF, rf_body, zero)
            for b in range(B):
                o_v[pl.ds(o_base0 + b * OC, OC)] = (
                    o_v[pl.ds(o_base0 + b * OC, OC)] + sw0)

        pltpu.sync_copy(o_v, out_hbm.at[pl.ds(wid * OUT_W, OUT_W)])


def kernel(x, input_mask, lut_weights):
    xf = x.reshape(-1)
    flat = (input_mask[:, 0] * (H * W) + input_mask[:, 1] * W
            + input_mask[:, 2]).astype(jnp.int32)
    out = _lutconv_sc(xf, flat, lut_weights.reshape(-1))
    out = out.reshape(S, B, OC)
    return out.transpose(1, 2, 0).reshape(B, OC, HO, WO)


# speedup vs baseline: 27.6439x; 1.6628x over previous
import functools

import jax
import jax.numpy as jnp
from jax import lax
from jax.experimental import pallas as pl
from jax.experimental.pallas import tpu as pltpu
from jax.experimental.pallas import tpu_sc as plsc

C_IN = 8
H = 16
W = 16
KH = 3
KW = 3
OC = 16
K = 2
HO = H - KH + 1
WO = W - KW + 1
S = HO * WO
N_RF = C_IN * KH * KW
B = 32
T = OC * S * N_RF

NC = 2
NS = 16
NW = 28
S_PER_W = S // NW

XLEN = B * C_IN * H * W
ROWS_OC = S_PER_W * N_RF
CI_OC = ROWS_OC * K
WT_W = 4 * OC * ROWS_OC
OUT_W = S_PER_W * B * OC

_mesh = plsc.VectorSubcoreMesh(core_axis_name="c", subcore_axis_name="s")


@functools.partial(
    pl.kernel,
    mesh=_mesh,
    compiler_params=pltpu.CompilerParams(needs_layout_passes=False),
    out_type=jax.ShapeDtypeStruct((S * B * OC,), jnp.float32),
    scratch_types=[
        pltpu.VMEM((XLEN,), jnp.float32),
        pltpu.VMEM((OC * CI_OC,), jnp.int32),
        pltpu.VMEM((WT_W,), jnp.float32),
        pltpu.VMEM((OUT_W,), jnp.float32),
        pltpu.SemaphoreType.DMA,
    ],
)
def _lutconv_sc(x_hbm, ci_hbm, wt_hbm, out_hbm, x_v, ci_v, wt_v, o_v, sem):
    wid = lax.axis_index("s") * NC + lax.axis_index("c")

    @pl.when(wid < NW)
    def _body():
        copies = [pltpu.async_copy(x_hbm, x_v, sem)]
        for oc in range(OC):
            copies.append(pltpu.async_copy(
                ci_hbm.at[pl.ds(oc * (S * N_RF * K) + wid * CI_OC, CI_OC)],
                ci_v.at[pl.ds(oc * CI_OC, CI_OC)], sem))
        for j in range(4):
            for oc in range(OC):
                copies.append(pltpu.async_copy(
                    wt_hbm.at[pl.ds(j * T + oc * (S * N_RF) + wid * ROWS_OC,
                                    ROWS_OC)],
                    wt_v.at[pl.ds((j * OC + oc) * ROWS_OC, ROWS_OC)], sem))
        for h in copies:
            h.wait()

        zero = jnp.zeros((OC,), jnp.float32)
        iota = lax.iota(jnp.int32, OC)
        ioc_ci = iota * CI_OC
        ioc_row = iota * ROWS_OC

        for si in range(S_PER_W):
            o_base0 = si * B * OC

            for b in range(B):
                o_v[pl.ds(o_base0 + b * OC, OC)] = zero

            def rf_body(rf, sw0, si=si, o_base0=o_base0):
                cib = ioc_ci + (si * (N_RF * K) + rf * K)
                rv = ioc_row + (si * N_RF + rf)
                ci0 = plsc.load_gather(ci_v, [cib])
                ci1 = plsc.load_gather(ci_v, [cib + 1])
                w0 = plsc.load_gather(wt_v, [rv])
                w1 = plsc.load_gather(wt_v, [rv + (OC * ROWS_OC)])
                w2 = plsc.load_gather(wt_v, [rv + 2 * (OC * ROWS_OC)])
                w3 = plsc.load_gather(wt_v, [rv + 3 * (OC * ROWS_OC)])
                bb = w2 - w0
                cc = w1 - w0
                aa = (w3 + w0) - (w1 + w2)
                for b in range(B):
                    off = b * (C_IN * H * W)
                    p0 = plsc.load_gather(x_v, [ci0 + off])
                    p1 = plsc.load_gather(x_v, [ci1 + off])
                    v = p0 * bb + (p1 * cc + (p0 * p1) * aa)
                    plsc.addupdate(o_v.at[pl.ds(o_base0 + b * OC, OC)], v)
                return sw0 + w0

            sw0 = lax.fori_loop(0, N_RF, rf_body, zero)
            for b in range(B):
                o_v[pl.ds(o_base0 + b * OC, OC)] = (
                    o_v[pl.ds(o_base0 + b * OC, OC)] + sw0)

        pltpu.sync_copy(o_v, out_hbm.at[pl.ds(wid * OUT_W, OUT_W)])


def kernel(x, input_mask, lut_weights):
    xf = x.reshape(-1)
    flat = (input_mask[:, 0] * (H * W) + input_mask[:, 1] * W
            + input_mask[:, 2]).astype(jnp.int32)
    wt_cols = lut_weights.T.reshape(-1)
    out = _lutconv_sc(xf, flat, wt_cols)
    out = out.reshape(S, B, OC)
    return out.transpose(1, 2, 0).reshape(B, OC, HO, WO)
